# baseline (speedup 1.0000x reference)
import functools

import jax
import jax.numpy as jnp
from jax import lax
from jax.experimental import pallas as pl
from jax.experimental.pallas import tpu as pltpu
from jax.experimental.pallas import tpu_sc as plsc

HASH_N = 20480
EMB = 128
DM = 2048
P1C = 31337

BATCH = 4
SEQ = 4096
NTOK = BATCH * SEQ
NWORK = 32

S = 4
TOK_S = NTOK // S
CHUNK = TOK_S // NWORK
GROUPS = CHUNK // 16
ROWS_PER_DMA = 128
NDMA = max(1, CHUNK // ROWS_PER_DMA)

CT = 1024
NBUF = 4
NC_S = TOK_S // CT


def _sc_gather_kernel(ids_hbm, table_hbm, emb_hbm, ids_v, h_v, rows_v, sem):
    wid = lax.axis_index("s") * 2 + lax.axis_index("c")
    base = wid * CHUNK

    @pl.when(wid % (SEQ // CHUNK) == 0)
    def _():
        ids_v[pl.ds(0, 16)] = jnp.zeros((16,), jnp.int32)
        pltpu.sync_copy(ids_hbm.at[pl.ds(base, CHUNK)], ids_v.at[pl.ds(8, CHUNK)])

    @pl.when(wid % (SEQ // CHUNK) != 0)
    def _():
        pltpu.sync_copy(ids_hbm.at[pl.ds(base - 8, CHUNK + 8)], ids_v)

    for g in range(GROUPS):
        cur = ids_v[pl.ds(8 + g * 16, 16)]
        prev = ids_v[pl.ds(7 + g * 16, 16)]
        x = (prev * P1C) ^ cur
        r = lax.rem(x, HASH_N)
        h = jnp.where(r < 0, r + HASH_N, r)
        h_v[g // (ROWS_PER_DMA // 16), pl.ds((g % (ROWS_PER_DMA // 16)) * 16, 16)] = h

    cps = [
        pltpu.async_copy(
            table_hbm.at[h_v.at[j]],
            rows_v.at[pl.ds(j * ROWS_PER_DMA, ROWS_PER_DMA)],
            sem,
        )
        for j in range(NDMA)
    ]
    for cp in cps:
        cp.wait()
    pltpu.sync_copy(rows_v, emb_hbm.at[pl.ds(base, CHUNK)])


def _sc_gather(ids_slice, table):
    mesh = plsc.VectorSubcoreMesh(core_axis_name="c", subcore_axis_name="s")
    fn = functools.partial(
        pl.kernel,
        mesh=mesh,
        out_type=jax.ShapeDtypeStruct((TOK_S, EMB), jnp.float32),
        scratch_types=[
            pltpu.VMEM((CHUNK + 8,), jnp.int32),
            pltpu.VMEM((NDMA, ROWS_PER_DMA), jnp.int32),
            pltpu.VMEM((CHUNK, EMB), jnp.float32),
            pltpu.SemaphoreType.DMA,
        ],
    )(_sc_gather_kernel)
    return fn(ids_slice, table)


def _mm_slice_body(x_ref, w_ref, *rest, s):
    o_hbm, ob, sems = rest[-3], rest[-2], rest[-1]
    w = w_ref[...].astype(jnp.bfloat16)
    cps = [None] * NBUF
    for j in range(NC_S):
        b = j % NBUF
        if cps[b] is not None:
            cps[b].wait()
        xj = x_ref[pl.ds(j * CT, CT), :].astype(jnp.bfloat16)
        ob[b] = lax.dot_general(
            xj, w,
            dimension_numbers=(((1,), (1,)), ((), ())),
            preferred_element_type=jnp.float32,
        )
        cps[b] = pltpu.make_async_copy(
            ob.at[b], o_hbm.at[pl.ds(s * TOK_S + j * CT, CT), :], sems.at[b]
        )
        cps[b].start()
    for b in range(NBUF):
        cps[b].wait()


def _project_slice(emb_s, proj_w, out, s):
    body = functools.partial(_mm_slice_body, s=s)
    vmem = pl.BlockSpec(memory_space=pltpu.MemorySpace.VMEM)
    hbm = pl.BlockSpec(memory_space=pltpu.MemorySpace.HBM)
    scratch = [
        pltpu.VMEM((NBUF, CT, DM), jnp.float32),
        pltpu.SemaphoreType.DMA((NBUF,)),
    ]
    if out is None:
        return pl.pallas_call(
            body,
            in_specs=[vmem, vmem],
            out_specs=hbm,
            out_shape=jax.ShapeDtypeStruct((NTOK, DM), jnp.float32),
            scratch_shapes=scratch,
        )(emb_s, proj_w)
    return pl.pallas_call(
        body,
        in_specs=[vmem, vmem, hbm],
        out_specs=hbm,
        out_shape=jax.ShapeDtypeStruct((NTOK, DM), jnp.float32),
        scratch_shapes=scratch,
        input_output_aliases={2: 0},
    )(emb_s, proj_w, out)


@jax.jit
def kernel(input_ids, bigram_emb, proj_w):
    ids_flat = input_ids.reshape(-1)
    embs = [
        _sc_gather(ids_flat[s * TOK_S : (s + 1) * TOK_S], bigram_emb)
        for s in range(S)
    ]
    out = None
    for s in range(S):
        out = _project_slice(embs[s], proj_w, out, s)
    return out.reshape(BATCH, SEQ, DM)


# speedup vs baseline: 1.0564x; 1.0564x over previous
import functools

import jax
import jax.numpy as jnp
from jax import lax
from jax.experimental import pallas as pl
from jax.experimental.pallas import tpu as pltpu
from jax.experimental.pallas import tpu_sc as plsc

HASH_N = 20480
EMB = 128
DM = 2048
P1C = 31337

BATCH = 4
SEQ = 4096
NTOK = BATCH * SEQ
NWORK = 32

S = 1
TOK_S = NTOK // S
CHUNK = TOK_S // NWORK
GROUPS = CHUNK // 16
ROWS_PER_DMA = 128
NDMA = max(1, CHUNK // ROWS_PER_DMA)

CT = 512
NBUF = 8
NC_S = TOK_S // CT


def _sc_gather_kernel(ids_hbm, table_hbm, emb_hbm, ids_v, h_v, rows_v, sem):
    wid = lax.axis_index("s") * 2 + lax.axis_index("c")
    base = wid * CHUNK

    @pl.when(wid % (SEQ // CHUNK) == 0)
    def _():
        ids_v[pl.ds(0, 16)] = jnp.zeros((16,), jnp.int32)
        pltpu.sync_copy(ids_hbm.at[pl.ds(base, CHUNK)], ids_v.at[pl.ds(8, CHUNK)])

    @pl.when(wid % (SEQ // CHUNK) != 0)
    def _():
        pltpu.sync_copy(ids_hbm.at[pl.ds(base - 8, CHUNK + 8)], ids_v)

    for g in range(GROUPS):
        cur = ids_v[pl.ds(8 + g * 16, 16)]
        prev = ids_v[pl.ds(7 + g * 16, 16)]
        x = (prev * P1C) ^ cur
        r = lax.rem(x, HASH_N)
        h = jnp.where(r < 0, r + HASH_N, r)
        h_v[g // (ROWS_PER_DMA // 16), pl.ds((g % (ROWS_PER_DMA // 16)) * 16, 16)] = h

    cps = [
        pltpu.async_copy(
            table_hbm.at[h_v.at[j]],
            rows_v.at[pl.ds(j * ROWS_PER_DMA, ROWS_PER_DMA)],
            sem,
        )
        for j in range(NDMA)
    ]
    for cp in cps:
        cp.wait()
    pltpu.sync_copy(rows_v, emb_hbm.at[pl.ds(base, CHUNK)])


def _sc_gather(ids_slice, table):
    mesh = plsc.VectorSubcoreMesh(core_axis_name="c", subcore_axis_name="s")
    fn = functools.partial(
        pl.kernel,
        mesh=mesh,
        out_type=jax.ShapeDtypeStruct((TOK_S, EMB), jnp.float32),
        scratch_types=[
            pltpu.VMEM((CHUNK + 8,), jnp.int32),
            pltpu.VMEM((NDMA, ROWS_PER_DMA), jnp.int32),
            pltpu.VMEM((CHUNK, EMB), jnp.float32),
            pltpu.SemaphoreType.DMA,
        ],
    )(_sc_gather_kernel)
    return fn(ids_slice, table)


def _mm_slice_body(x_ref, w_ref, *rest, s):
    o_hbm, ob, sems = rest[-3], rest[-2], rest[-1]
    w = w_ref[...].astype(jnp.bfloat16)
    cps = [None] * NBUF
    for j in range(NC_S):
        b = j % NBUF
        if cps[b] is not None:
            cps[b].wait()
        xj = x_ref[pl.ds(j * CT, CT), :].astype(jnp.bfloat16)
        ob[b] = lax.dot_general(
            xj, w,
            dimension_numbers=(((1,), (1,)), ((), ())),
            preferred_element_type=jnp.float32,
        )
        cps[b] = pltpu.make_async_copy(
            ob.at[b], o_hbm.at[pl.ds(s * TOK_S + j * CT, CT), :], sems.at[b]
        )
        cps[b].start()
    for b in range(NBUF):
        cps[b].wait()


def _project_slice(emb_s, proj_w, out, s):
    body = functools.partial(_mm_slice_body, s=s)
    vmem = pl.BlockSpec(memory_space=pltpu.MemorySpace.VMEM)
    hbm = pl.BlockSpec(memory_space=pltpu.MemorySpace.HBM)
    scratch = [
        pltpu.VMEM((NBUF, CT, DM), jnp.float32),
        pltpu.SemaphoreType.DMA((NBUF,)),
    ]
    if out is None:
        return pl.pallas_call(
            body,
            in_specs=[vmem, vmem],
            out_specs=hbm,
            out_shape=jax.ShapeDtypeStruct((NTOK, DM), jnp.float32),
            scratch_shapes=scratch,
        )(emb_s, proj_w)
    return pl.pallas_call(
        body,
        in_specs=[vmem, vmem, hbm],
        out_specs=hbm,
        out_shape=jax.ShapeDtypeStruct((NTOK, DM), jnp.float32),
        scratch_shapes=scratch,
        input_output_aliases={2: 0},
    )(emb_s, proj_w, out)


@jax.jit
def kernel(input_ids, bigram_emb, proj_w):
    ids_flat = input_ids.reshape(-1)
    embs = [
        _sc_gather(ids_flat[s * TOK_S : (s + 1) * TOK_S], bigram_emb)
        for s in range(S)
    ]
    out = None
    for s in range(S):
        out = _project_slice(embs[s], proj_w, out, s)
    return out.reshape(BATCH, SEQ, DM)


# speedup vs baseline: 1.0740x; 1.0166x over previous
import functools

import jax
import jax.numpy as jnp
from jax import lax
from jax.experimental import pallas as pl
from jax.experimental.pallas import tpu as pltpu
from jax.experimental.pallas import tpu_sc as plsc

HASH_N = 20480
EMB = 128
DM = 2048
P1C = 31337

BATCH = 4
SEQ = 4096
NTOK = BATCH * SEQ
NWORK = 32

S = 2
TOK_S = NTOK // S
CHUNK = TOK_S // NWORK
GROUPS = CHUNK // 16
ROWS_PER_DMA = 128
NDMA = max(1, CHUNK // ROWS_PER_DMA)

CT = 512
NBUF = 6
NC_S = TOK_S // CT


def _sc_gather_kernel(ids_hbm, table_hbm, emb_hbm, ids_v, h_v, rows_v, sem):
    wid = lax.axis_index("s") * 2 + lax.axis_index("c")
    base = wid * CHUNK

    @pl.when(wid % (SEQ // CHUNK) == 0)
    def _():
        ids_v[pl.ds(0, 16)] = jnp.zeros((16,), jnp.int32)
        pltpu.sync_copy(ids_hbm.at[pl.ds(base, CHUNK)], ids_v.at[pl.ds(8, CHUNK)])

    @pl.when(wid % (SEQ // CHUNK) != 0)
    def _():
        pltpu.sync_copy(ids_hbm.at[pl.ds(base - 8, CHUNK + 8)], ids_v)

    for g in range(GROUPS):
        cur = ids_v[pl.ds(8 + g * 16, 16)]
        prev = ids_v[pl.ds(7 + g * 16, 16)]
        x = (prev * P1C) ^ cur
        r = lax.rem(x, HASH_N)
        h = jnp.where(r < 0, r + HASH_N, r)
        h_v[g // (ROWS_PER_DMA // 16), pl.ds((g % (ROWS_PER_DMA // 16)) * 16, 16)] = h

    cps = [
        pltpu.async_copy(
            table_hbm.at[h_v.at[j]],
            rows_v.at[pl.ds(j * ROWS_PER_DMA, ROWS_PER_DMA)],
            sem,
        )
        for j in range(NDMA)
    ]
    for cp in cps:
        cp.wait()
    pltpu.sync_copy(rows_v, emb_hbm.at[pl.ds(base, CHUNK)])


def _sc_gather(ids_slice, table):
    mesh = plsc.VectorSubcoreMesh(core_axis_name="c", subcore_axis_name="s")
    fn = functools.partial(
        pl.kernel,
        mesh=mesh,
        out_type=jax.ShapeDtypeStruct((TOK_S, EMB), jnp.float32),
        scratch_types=[
            pltpu.VMEM((CHUNK + 8,), jnp.int32),
            pltpu.VMEM((NDMA, ROWS_PER_DMA), jnp.int32),
            pltpu.VMEM((CHUNK, EMB), jnp.float32),
            pltpu.SemaphoreType.DMA,
        ],
    )(_sc_gather_kernel)
    return fn(ids_slice, table)


def _mm_slice_body(x_ref, w_ref, *rest, s):
    o_hbm, ob, sems = rest[-3], rest[-2], rest[-1]
    w = w_ref[...].astype(jnp.bfloat16)
    cps = [None] * NBUF
    for j in range(NC_S):
        b = j % NBUF
        if cps[b] is not None:
            cps[b].wait()
        xj = x_ref[pl.ds(j * CT, CT), :].astype(jnp.bfloat16)
        ob[b] = lax.dot_general(
            xj, w,
            dimension_numbers=(((1,), (1,)), ((), ())),
            preferred_element_type=jnp.float32,
        )
        cps[b] = pltpu.make_async_copy(
            ob.at[b], o_hbm.at[pl.ds(s * TOK_S + j * CT, CT), :], sems.at[b]
        )
        cps[b].start()
    for b in range(NBUF):
        cps[b].wait()


def _project_slice(emb_s, proj_w, out, s):
    body = functools.partial(_mm_slice_body, s=s)
    vmem = pl.BlockSpec(memory_space=pltpu.MemorySpace.VMEM)
    hbm = pl.BlockSpec(memory_space=pltpu.MemorySpace.HBM)
    scratch = [
        pltpu.VMEM((NBUF, CT, DM), jnp.float32),
        pltpu.SemaphoreType.DMA((NBUF,)),
    ]
    if out is None:
        return pl.pallas_call(
            body,
            in_specs=[vmem, vmem],
            out_specs=hbm,
            out_shape=jax.ShapeDtypeStruct((NTOK, DM), jnp.float32),
            scratch_shapes=scratch,
        )(emb_s, proj_w)
    return pl.pallas_call(
        body,
        in_specs=[vmem, vmem, hbm],
        out_specs=hbm,
        out_shape=jax.ShapeDtypeStruct((NTOK, DM), jnp.float32),
        scratch_shapes=scratch,
        input_output_aliases={2: 0},
    )(emb_s, proj_w, out)


@jax.jit
def kernel(input_ids, bigram_emb, proj_w):
    ids_flat = input_ids.reshape(-1)
    embs = [
        _sc_gather(ids_flat[s * TOK_S : (s + 1) * TOK_S], bigram_emb)
        for s in range(S)
    ]
    out = None
    for s in range(S):
        out = _project_slice(embs[s], proj_w, out, s)
    return out.reshape(BATCH, SEQ, DM)
